# Initial kernel scaffold; baseline (speedup 1.0000x reference)
import jax
import jax.numpy as jnp
from jax.experimental import pallas as pl


def kernel(code_x, divided, neighbors, lens, adj, c_emb, n_emb, u_emb, Wg, bg, W_ih, b_ih, W_hh, b_hh, Wq, bq, Wk, bk, Wv, bv, Wd, bd, ctx, Wc, bc):
    raise NotImplementedError("write your pallas kernel here")


# speedup vs baseline: 3.0456x; 3.0456x over previous
import functools

import jax
import jax.numpy as jnp
from jax.experimental import pallas as pl

NEG = -1e9
NP = 1024
HP = 152


def _body(maskp_ref, selrow_ref, vneg_ref, adj_ref, ce_ref, ne_ref, ue_ref,
          wg_ref, bg_ref,
          wir_ref, wiz_ref, win_ref, bir_ref, biz_ref, bin_ref,
          whr_ref, whz_ref, whn_ref, bhr_ref, bhz_ref, bhn_ref,
          wq_ref, bq_ref, wk_ref, bk_ref, wv_ref, bv_ref,
          wd_ref, bd_ref, ctx_ref, wc_ref, bc_ref, out_ref,
          *, T, GS, OUT, inv_sqrt_ta):
    f32 = jnp.float32
    adjm = adj_ref[...]
    cemb = ce_ref[...]
    nemb = ne_ref[...]
    uemb = ue_ref[...]
    wg = wg_ref[...]; bg = bg_ref[...]
    wir = wir_ref[...]; wiz = wiz_ref[...]; win = win_ref[...]
    bir = bir_ref[...]; biz = biz_ref[...]; bin_ = bin_ref[...]
    whr = whr_ref[...]; whz = whz_ref[...]; whn = whn_ref[...]
    bhr = bhr_ref[...]; bhz = bhz_ref[...]; bhn = bhn_ref[...]
    wq = wq_ref[...]; bq = bq_ref[...]
    wk = wk_ref[...]; bk = bk_ref[...]
    wv = wv_ref[...]; bv = bv_ref[...]

    h = jnp.zeros((NP, HP), f32)
    noprev = jnp.zeros((NP, GS), f32)
    rows = []
    for t in range(T):
        mm = maskp_ref[0, t]
        c = mm[:, 0:1]
        n = mm[:, 1:2]
        m1 = mm[:, 2:3]
        m2 = mm[:, 3:4]
        m3 = mm[:, 4:5]

        x0 = c * cemb + n * nemb
        s = jnp.dot(adjm, x0, preferred_element_type=f32)

        cg = jnp.dot(c * (cemb + s), wg, preferred_element_type=f32) + bg
        co = jnp.where(cg >= 0, cg, 0.01 * cg)
        ngi = jnp.dot(n * (nemb + s), wg, preferred_element_type=f32) + bg
        no = jnp.where(ngi >= 0, ngi, 0.01 * ngi)

        gi_r = jnp.dot(co, wir, preferred_element_type=f32) + bir
        gi_z = jnp.dot(co, wiz, preferred_element_type=f32) + biz
        gi_n = jnp.dot(co, win, preferred_element_type=f32) + bin_
        gh_r = jnp.dot(h, whr, preferred_element_type=f32) + bhr
        gh_z = jnp.dot(h, whz, preferred_element_type=f32) + bhz
        gh_n = jnp.dot(h, whn, preferred_element_type=f32) + bhn
        r = jax.nn.sigmoid(gi_r + gh_r)
        z = jax.nn.sigmoid(gi_z + gh_z)
        ng = jnp.tanh(gi_n + r * gh_n)
        h_all = (1.0 - z) * ng + z * h

        hnew = m1 * h_all
        anym1 = jnp.max(m1) > 0.0
        mx1 = jnp.max(jnp.where(m1 > 0, h_all, NEG), axis=0, keepdims=True)
        outm1 = jnp.where(anym1, mx1, 0.0)

        if t == 0:
            outrow = outm1
        else:
            sel = jnp.maximum(m2, m3)
            srow = selrow_ref[0, t:t + 1, :]
            qsrc = m2 * noprev + (1.0 - m2) * (m3 * uemb)
            query = jnp.dot(qsrc, wq, preferred_element_type=f32) + bq
            key = jnp.dot(qsrc, wk, preferred_element_type=f32) + bk
            value = jnp.dot(co, wv, preferred_element_type=f32) + bv
            g = jax.lax.dot_general(
                query, key, (((1,), (1,)), ((), ())),
                preferred_element_type=f32) * inv_sqrt_ta
            g = jnp.where(srow > 0, g, NEG)
            gm = jnp.max(g, axis=1, keepdims=True)
            e = jnp.exp(g - gm)
            score = e / jnp.sum(e, axis=1, keepdims=True)
            hm = jnp.tanh(jnp.dot(score, value, preferred_element_type=f32))
            hnew = sel * hm + (1.0 - sel) * hnew
            anysel = jnp.max(sel) > 0.0
            mx23 = jnp.max(jnp.where(sel > 0, hm, NEG), axis=0, keepdims=True)
            outm23 = jnp.where(anysel, mx23, 0.0)
            outrow = outm1 + outm23
        rows.append(outrow)
        h = hnew
        noprev = no

    X = jnp.concatenate(rows, axis=0)
    tt = jnp.dot(X, wd_ref[...], preferred_element_type=f32) + bd_ref[...]
    vu = jnp.dot(tt, ctx_ref[...], preferred_element_type=f32)
    vv = vu + vneg_ref[0]
    vm = jnp.max(vv, axis=0, keepdims=True)
    ev = jnp.exp(vv - vm)
    sc = ev / jnp.sum(ev, axis=0, keepdims=True)
    pooled = jnp.sum(X * sc, axis=0, keepdims=True)
    res = jnp.dot(pooled, wc_ref[...], preferred_element_type=f32) + bc_ref[...]
    out_ref[...] = jnp.broadcast_to(res[None], (1, 8, OUT))


def kernel(code_x, divided, neighbors, lens, adj, c_emb, n_emb, u_emb,
           Wg, bg, W_ih, b_ih, W_hh, b_hh, Wq, bq, Wk, bk, Wv, bv,
           Wd, bd, ctx, Wc, bc):
    f32 = jnp.float32
    B, T, N = code_x.shape
    CS = c_emb.shape[1]
    GS = Wg.shape[1]
    TA = Wq.shape[1]
    HS = W_hh.shape[1]
    OUT = Wc.shape[1]
    pn = NP - N
    ph = HP - HS

    adj_p = jnp.pad(adj, ((0, pn), (0, pn)))
    ce_p = jnp.pad(c_emb, ((0, pn), (0, 0)))
    ne_p = jnp.pad(n_emb, ((0, pn), (0, 0)))
    ue_p = jnp.pad(u_emb, ((0, pn), (0, 0)))

    m = (divided > 0).astype(f32)
    maskp = jnp.concatenate(
        [code_x[..., None], neighbors[..., None], m], axis=-1)
    maskp = jnp.pad(maskp, ((0, 0), (0, 0), (0, pn), (0, 3)))
    selrow = jnp.pad(jnp.maximum(m[..., 1], m[..., 2]),
                     ((0, 0), (0, 0), (0, pn)))

    lens_i = jnp.maximum(jnp.asarray(lens).astype(jnp.int32), 1)
    valid = jnp.arange(T)[None, :] < lens_i[:, None]
    vneg = jnp.where(valid, 0.0, NEG).astype(f32)[..., None]

    wihT = W_ih.T
    whhT = W_hh.T
    def _split_i(k):
        return jnp.pad(wihT[:, k * HS:(k + 1) * HS], ((0, 0), (0, ph)))
    def _split_h(k):
        return jnp.pad(whhT[:, k * HS:(k + 1) * HS], ((0, ph), (0, ph)))
    def _split_b(b, k):
        return jnp.pad(b[k * HS:(k + 1) * HS], (0, ph))[None]
    wir, wiz, win = _split_i(0), _split_i(1), _split_i(2)
    whr, whz, whn = _split_h(0), _split_h(1), _split_h(2)
    bir, biz, bin_ = _split_b(b_ih, 0), _split_b(b_ih, 1), _split_b(b_ih, 2)
    bhr, bhz, bhn = _split_b(b_hh, 0), _split_b(b_hh, 1), _split_b(b_hh, 2)

    wv_p = jnp.pad(Wv, ((0, 0), (0, ph)))
    bv_p = jnp.pad(bv, (0, ph))[None]
    wd_p = jnp.pad(Wd, ((0, ph), (0, 0)))
    wc_p = jnp.pad(Wc, ((0, ph), (0, 0)))
    ctx_c = ctx[:, None]

    operands = (
        maskp, selrow, vneg, adj_p, ce_p, ne_p, ue_p,
        Wg, bg[None],
        wir, wiz, win, bir, biz, bin_,
        whr, whz, whn, bhr, bhz, bhn,
        Wq, bq[None], Wk, bk[None], wv_p, bv_p,
        wd_p, bd[None], ctx_c, wc_p, bc[None],
    )

    def _spec(arr, batched):
        nd = arr.ndim
        if batched:
            blk = (1,) + arr.shape[1:]
            return pl.BlockSpec(blk, lambda i: (i,) + (0,) * (nd - 1))
        return pl.BlockSpec(arr.shape, lambda i: (0,) * nd)

    in_specs = [_spec(a, j < 3) for j, a in enumerate(operands)]
    out_specs = pl.BlockSpec((1, 8, OUT), lambda i: (i, 0, 0))

    body = functools.partial(_body, T=T, GS=GS, OUT=OUT,
                             inv_sqrt_ta=float(1.0 / (TA ** 0.5)))
    out = pl.pallas_call(
        body,
        grid=(B,),
        in_specs=in_specs,
        out_specs=out_specs,
        out_shape=jax.ShapeDtypeStruct((B, 8, OUT), f32),
    )(*operands)
    return out[:, 0, :]


# speedup vs baseline: 3.9542x; 1.2984x over previous
import functools

import jax
import jax.numpy as jnp
from jax.experimental import pallas as pl

NEG = -1e9
NP = 1024
HP = 152
KC = 256


def _body(*refs, T, GS, OUT, inv_sqrt_ta, compact):
    f32 = jnp.float32
    if compact:
        (maskp_ref, selrow_ref, vneg_ref, ut_ref, adj_ref, ce_ref, ne_ref,
         ue_ref, wg_ref, bg_ref,
         wir_ref, wiz_ref, win_ref, bir_ref, biz_ref, bin_ref,
         whr_ref, whz_ref, whn_ref, bhr_ref, bhz_ref, bhn_ref,
         wq_ref, bq_ref, wk_ref, bk_ref, wv_ref, bv_ref,
         wd_ref, bd_ref, ctx_ref, wc_ref, bc_ref, out_ref) = refs
    else:
        (maskp_ref, selrow_ref, vneg_ref, adj_ref, ce_ref, ne_ref,
         ue_ref, wg_ref, bg_ref,
         wir_ref, wiz_ref, win_ref, bir_ref, biz_ref, bin_ref,
         whr_ref, whz_ref, whn_ref, bhr_ref, bhz_ref, bhn_ref,
         wq_ref, bq_ref, wk_ref, bk_ref, wv_ref, bv_ref,
         wd_ref, bd_ref, ctx_ref, wc_ref, bc_ref, out_ref) = refs

    adjm = adj_ref[...]
    cemb = ce_ref[...]
    nemb = ne_ref[...]
    uemb = ue_ref[...]
    wg = wg_ref[...]; bg = bg_ref[...]
    wir = wir_ref[...]; wiz = wiz_ref[...]; win = win_ref[...]
    bir = bir_ref[...]; biz = biz_ref[...]; bin_ = bin_ref[...]
    whr = whr_ref[...]; whz = whz_ref[...]; whn = whn_ref[...]
    bhr = bhr_ref[...]; bhz = bhz_ref[...]; bhn = bhn_ref[...]
    wq = wq_ref[...]; bq = bq_ref[...]
    wk = wk_ref[...]; bk = bk_ref[...]
    wv = wv_ref[...]; bv = bv_ref[...]

    if compact:
        srow_all = selrow_ref[0]
        pos_all = jnp.dot(srow_all, ut_ref[...],
                          preferred_element_type=f32)
        counts = pos_all[:, NP - 1:NP]
        kio_col = jax.lax.broadcasted_iota(jnp.int32, (KC, 1), 0).astype(f32)
        kio_row = jax.lax.broadcasted_iota(jnp.int32, (1, KC), 1).astype(f32)

    h = jnp.zeros((NP, HP), f32)
    noprev = jnp.zeros((NP, GS), f32)
    rows = []
    for t in range(T):
        mm = maskp_ref[0, t]
        c = mm[:, 0:1]
        n = mm[:, 1:2]
        m1 = mm[:, 2:3]
        m2 = mm[:, 3:4]
        m3 = mm[:, 4:5]

        x0 = c * cemb + n * nemb
        s = jnp.dot(adjm, x0, preferred_element_type=f32)

        cg = jnp.dot(c * (cemb + s), wg, preferred_element_type=f32) + bg
        co = jnp.where(cg >= 0, cg, 0.01 * cg)
        ngi = jnp.dot(n * (nemb + s), wg, preferred_element_type=f32) + bg
        no = jnp.where(ngi >= 0, ngi, 0.01 * ngi)

        gi_r = jnp.dot(co, wir, preferred_element_type=f32) + bir
        gi_z = jnp.dot(co, wiz, preferred_element_type=f32) + biz
        gi_n = jnp.dot(co, win, preferred_element_type=f32) + bin_
        gh_r = jnp.dot(h, whr, preferred_element_type=f32) + bhr
        gh_z = jnp.dot(h, whz, preferred_element_type=f32) + bhz
        gh_n = jnp.dot(h, whn, preferred_element_type=f32) + bhn
        r = jax.nn.sigmoid(gi_r + gh_r)
        z = jax.nn.sigmoid(gi_z + gh_z)
        ng = jnp.tanh(gi_n + r * gh_n)
        h_all = (1.0 - z) * ng + z * h

        hnew = m1 * h_all
        anym1 = jnp.max(m1) > 0.0
        mx1 = jnp.max(jnp.where(m1 > 0, h_all, NEG), axis=0, keepdims=True)
        outm1 = jnp.where(anym1, mx1, 0.0)

        if t == 0:
            outrow = outm1
        else:
            sel = jnp.maximum(m2, m3)
            srow = selrow_ref[0, t:t + 1, :]
            qsrc = m2 * noprev + (1.0 - m2) * (m3 * uemb)
            query = jnp.dot(qsrc, wq, preferred_element_type=f32) + bq
            if compact:
                pos_t = pos_all[t:t + 1, :]
                onehot = jnp.where(
                    (pos_t == kio_col + 1.0) & (srow > 0), 1.0, 0.0)
                qc = jnp.dot(onehot, qsrc, preferred_element_type=f32)
                coc = jnp.dot(onehot, co, preferred_element_type=f32)
                keyc = jnp.dot(qc, wk, preferred_element_type=f32) + bk
                valuec = jnp.dot(coc, wv, preferred_element_type=f32) + bv
                g = jax.lax.dot_general(
                    query, keyc, (((1,), (1,)), ((), ())),
                    preferred_element_type=f32) * inv_sqrt_ta
                g = jnp.where(kio_row < counts[t:t + 1, :], g, NEG)
                gm = jnp.max(g, axis=1, keepdims=True)
                e = jnp.exp(g - gm)
                score = e / jnp.sum(e, axis=1, keepdims=True)
                hm = jnp.tanh(
                    jnp.dot(score, valuec, preferred_element_type=f32))
            else:
                key = jnp.dot(qsrc, wk, preferred_element_type=f32) + bk
                value = jnp.dot(co, wv, preferred_element_type=f32) + bv
                g = jax.lax.dot_general(
                    query, key, (((1,), (1,)), ((), ())),
                    preferred_element_type=f32) * inv_sqrt_ta
                g = jnp.where(srow > 0, g, NEG)
                gm = jnp.max(g, axis=1, keepdims=True)
                e = jnp.exp(g - gm)
                score = e / jnp.sum(e, axis=1, keepdims=True)
                hm = jnp.tanh(
                    jnp.dot(score, value, preferred_element_type=f32))
            hnew = sel * hm + (1.0 - sel) * hnew
            anysel = jnp.max(sel) > 0.0
            mx23 = jnp.max(jnp.where(sel > 0, hm, NEG), axis=0, keepdims=True)
            outm23 = jnp.where(anysel, mx23, 0.0)
            outrow = outm1 + outm23
        rows.append(outrow)
        h = hnew
        noprev = no

    X = jnp.concatenate(rows, axis=0)
    tt = jnp.dot(X, wd_ref[...], preferred_element_type=f32) + bd_ref[...]
    vu = jnp.dot(tt, ctx_ref[...], preferred_element_type=f32)
    vv = vu + vneg_ref[0]
    vm = jnp.max(vv, axis=0, keepdims=True)
    ev = jnp.exp(vv - vm)
    sc = ev / jnp.sum(ev, axis=0, keepdims=True)
    pooled = jnp.sum(X * sc, axis=0, keepdims=True)
    res = jnp.dot(pooled, wc_ref[...], preferred_element_type=f32) + bc_ref[...]
    out_ref[...] = jnp.broadcast_to(res[None], (1, 8, OUT))


def _spec(arr, batched):
    nd = arr.ndim
    if batched:
        blk = (1,) + arr.shape[1:]
        return pl.BlockSpec(blk, lambda i, _n=nd: (i,) + (0,) * (_n - 1))
    return pl.BlockSpec(arr.shape, lambda i, _n=nd: (0,) * _n)


def _make_call(operands, B, OUT, body):
    in_specs = [_spec(a, j < 3) for j, a in enumerate(operands)]
    out_specs = pl.BlockSpec((1, 8, OUT), lambda i: (i, 0, 0))
    return pl.pallas_call(
        body,
        grid=(B,),
        in_specs=in_specs,
        out_specs=out_specs,
        out_shape=jax.ShapeDtypeStruct((B, 8, OUT), jnp.float32),
    )(*operands)


def kernel(code_x, divided, neighbors, lens, adj, c_emb, n_emb, u_emb,
           Wg, bg, W_ih, b_ih, W_hh, b_hh, Wq, bq, Wk, bk, Wv, bv,
           Wd, bd, ctx, Wc, bc):
    f32 = jnp.float32
    B, T, N = code_x.shape
    GS = Wg.shape[1]
    TA = Wq.shape[1]
    HS = W_hh.shape[1]
    OUT = Wc.shape[1]
    pn = NP - N
    ph = HP - HS

    adj_p = jnp.pad(adj, ((0, pn), (0, pn)))
    ce_p = jnp.pad(c_emb, ((0, pn), (0, 0)))
    ne_p = jnp.pad(n_emb, ((0, pn), (0, 0)))
    ue_p = jnp.pad(u_emb, ((0, pn), (0, 0)))

    m = (divided > 0).astype(f32)
    maskp = jnp.concatenate(
        [code_x[..., None], neighbors[..., None], m], axis=-1)
    maskp = jnp.pad(maskp, ((0, 0), (0, 0), (0, pn), (0, 3)))
    selrow_n = jnp.maximum(m[..., 1], m[..., 2])
    selrow = jnp.pad(selrow_n, ((0, 0), (0, 0), (0, pn)))

    lens_i = jnp.maximum(jnp.asarray(lens).astype(jnp.int32), 1)
    valid = jnp.arange(T)[None, :] < lens_i[:, None]
    vneg = jnp.where(valid, 0.0, NEG).astype(f32)[..., None]

    ut = (jnp.arange(NP)[:, None] <= jnp.arange(NP)[None, :]).astype(f32)

    wihT = W_ih.T
    whhT = W_hh.T
    def _split_i(k):
        return jnp.pad(wihT[:, k * HS:(k + 1) * HS], ((0, 0), (0, ph)))
    def _split_h(k):
        return jnp.pad(whhT[:, k * HS:(k + 1) * HS], ((0, ph), (0, ph)))
    def _split_b(b, k):
        return jnp.pad(b[k * HS:(k + 1) * HS], (0, ph))[None]
    wir, wiz, win = _split_i(0), _split_i(1), _split_i(2)
    whr, whz, whn = _split_h(0), _split_h(1), _split_h(2)
    bir, biz, bin_ = _split_b(b_ih, 0), _split_b(b_ih, 1), _split_b(b_ih, 2)
    bhr, bhz, bhn = _split_b(b_hh, 0), _split_b(b_hh, 1), _split_b(b_hh, 2)

    wv_p = jnp.pad(Wv, ((0, 0), (0, ph)))
    bv_p = jnp.pad(bv, (0, ph))[None]
    wd_p = jnp.pad(Wd, ((0, ph), (0, 0)))
    wc_p = jnp.pad(Wc, ((0, ph), (0, 0)))
    ctx_c = ctx[:, None]

    shared = (
        adj_p, ce_p, ne_p, ue_p,
        Wg, bg[None],
        wir, wiz, win, bir, biz, bin_,
        whr, whz, whn, bhr, bhz, bhn,
        Wq, bq[None], Wk, bk[None], wv_p, bv_p,
        wd_p, bd[None], ctx_c, wc_p, bc[None],
    )

    mk = functools.partial(_body, T=T, GS=GS, OUT=OUT,
                           inv_sqrt_ta=float(1.0 / (TA ** 0.5)))

    def _compact(ops):
        maskp_, selrow_, vneg_, ut_, rest = ops[0], ops[1], ops[2], ops[3], ops[4:]
        operands = (maskp_, selrow_, vneg_, ut_) + rest
        return _make_call(operands, B, OUT,
                          functools.partial(mk, compact=True))

    def _dense(ops):
        operands = (ops[0], ops[1], ops[2]) + ops[4:]
        return _make_call(operands, B, OUT,
                          functools.partial(mk, compact=False))

    fits = jnp.max(jnp.sum(selrow_n, axis=-1)) <= float(KC)
    out = jax.lax.cond(fits, _compact, _dense,
                       (maskp, selrow, vneg, ut) + shared)
    return out[:, 0, :]


# speedup vs baseline: 4.1485x; 1.0491x over previous
import functools

import jax
import jax.numpy as jnp
from jax.experimental import pallas as pl

NEG = -1e9
NP = 1024
HP = 152
KC = 128


def _body(*refs, T, GS, OUT, inv_sqrt_ta, compact):
    f32 = jnp.float32
    if compact:
        (maskp_ref, selrow_ref, vneg_ref, ut_ref, adj_ref, ce_ref, ne_ref,
         ue_ref, wg_ref, bg_ref,
         wir_ref, wiz_ref, win_ref, bir_ref, biz_ref, bin_ref,
         whr_ref, whz_ref, whn_ref, bhr_ref, bhz_ref, bhn_ref,
         wq_ref, bq_ref, wk_ref, bk_ref, wv_ref, bv_ref,
         wd_ref, bd_ref, ctx_ref, wc_ref, bc_ref, out_ref) = refs
    else:
        (maskp_ref, selrow_ref, vneg_ref, adj_ref, ce_ref, ne_ref,
         ue_ref, wg_ref, bg_ref,
         wir_ref, wiz_ref, win_ref, bir_ref, biz_ref, bin_ref,
         whr_ref, whz_ref, whn_ref, bhr_ref, bhz_ref, bhn_ref,
         wq_ref, bq_ref, wk_ref, bk_ref, wv_ref, bv_ref,
         wd_ref, bd_ref, ctx_ref, wc_ref, bc_ref, out_ref) = refs

    adjm = adj_ref[...]
    cemb = ce_ref[...]
    nemb = ne_ref[...]
    uemb = ue_ref[...]
    wg = wg_ref[...]; bg = bg_ref[...]
    wir = wir_ref[...]; wiz = wiz_ref[...]; win = win_ref[...]
    bir = bir_ref[...]; biz = biz_ref[...]; bin_ = bin_ref[...]
    whr = whr_ref[...]; whz = whz_ref[...]; whn = whn_ref[...]
    bhr = bhr_ref[...]; bhz = bhz_ref[...]; bhn = bhn_ref[...]
    wq = wq_ref[...]; bq = bq_ref[...]
    wk = wk_ref[...]; bk = bk_ref[...]
    wv = wv_ref[...]; bv = bv_ref[...]

    if compact:
        srow_all = selrow_ref[0]
        pos_all = jnp.dot(srow_all, ut_ref[...],
                          preferred_element_type=f32)
        counts = pos_all[:, NP - 1:NP]
        kio_col = jax.lax.broadcasted_iota(jnp.int32, (KC, 1), 0).astype(f32)
        kio_row = jax.lax.broadcasted_iota(jnp.int32, (1, KC), 1).astype(f32)

    CS = cemb.shape[1]
    msk = []
    x0s = []
    for t in range(T):
        mm = maskp_ref[0, t]
        cols = (mm[:, 0:1], mm[:, 1:2], mm[:, 2:3], mm[:, 3:4], mm[:, 4:5])
        msk.append(cols)
        x0s.append(cols[0] * cemb + cols[1] * nemb)
    s_all = jnp.dot(adjm, jnp.concatenate(x0s, axis=1),
                    preferred_element_type=f32)

    h = jnp.zeros((NP, HP), f32)
    noprev = jnp.zeros((NP, GS), f32)
    rows = []
    for t in range(T):
        c, n, m1, m2, m3 = msk[t]
        s = s_all[:, t * CS:(t + 1) * CS]

        cg = jnp.dot(c * (cemb + s), wg, preferred_element_type=f32) + bg
        co = jnp.where(cg >= 0, cg, 0.01 * cg)
        ngi = jnp.dot(n * (nemb + s), wg, preferred_element_type=f32) + bg
        no = jnp.where(ngi >= 0, ngi, 0.01 * ngi)

        gi_r = jnp.dot(co, wir, preferred_element_type=f32) + bir
        gi_z = jnp.dot(co, wiz, preferred_element_type=f32) + biz
        gi_n = jnp.dot(co, win, preferred_element_type=f32) + bin_
        gh_r = jnp.dot(h, whr, preferred_element_type=f32) + bhr
        gh_z = jnp.dot(h, whz, preferred_element_type=f32) + bhz
        gh_n = jnp.dot(h, whn, preferred_element_type=f32) + bhn
        r = jax.nn.sigmoid(gi_r + gh_r)
        z = jax.nn.sigmoid(gi_z + gh_z)
        ng = jnp.tanh(gi_n + r * gh_n)
        h_all = (1.0 - z) * ng + z * h

        hnew = m1 * h_all
        anym1 = jnp.max(m1) > 0.0
        mx1 = jnp.max(jnp.where(m1 > 0, h_all, NEG), axis=0, keepdims=True)
        outm1 = jnp.where(anym1, mx1, 0.0)

        if t == 0:
            outrow = outm1
        else:
            sel = jnp.maximum(m2, m3)
            srow = selrow_ref[0, t:t + 1, :]
            qsrc = m2 * noprev + (1.0 - m2) * (m3 * uemb)
            query = jnp.dot(qsrc, wq, preferred_element_type=f32) + bq
            if compact:
                pos_t = pos_all[t:t + 1, :]
                onehot = jnp.where(
                    (pos_t == kio_col + 1.0) & (srow > 0), 1.0, 0.0)
                qc = jnp.dot(onehot, qsrc, preferred_element_type=f32)
                coc = jnp.dot(onehot, co, preferred_element_type=f32)
                keyc = jnp.dot(qc, wk, preferred_element_type=f32) + bk
                valuec = jnp.dot(coc, wv, preferred_element_type=f32) + bv
                g = jax.lax.dot_general(
                    query, keyc, (((1,), (1,)), ((), ())),
                    preferred_element_type=f32) * inv_sqrt_ta
                g = jnp.where(kio_row < counts[t:t + 1, :], g, NEG)
                gm = jnp.max(g, axis=1, keepdims=True)
                e = jnp.exp(g - gm)
                score = e / jnp.sum(e, axis=1, keepdims=True)
                hm = jnp.tanh(
                    jnp.dot(score, valuec, preferred_element_type=f32))
            else:
                key = jnp.dot(qsrc, wk, preferred_element_type=f32) + bk
                value = jnp.dot(co, wv, preferred_element_type=f32) + bv
                g = jax.lax.dot_general(
                    query, key, (((1,), (1,)), ((), ())),
                    preferred_element_type=f32) * inv_sqrt_ta
                g = jnp.where(srow > 0, g, NEG)
                gm = jnp.max(g, axis=1, keepdims=True)
                e = jnp.exp(g - gm)
                score = e / jnp.sum(e, axis=1, keepdims=True)
                hm = jnp.tanh(
                    jnp.dot(score, value, preferred_element_type=f32))
            hnew = sel * hm + (1.0 - sel) * hnew
            anysel = jnp.max(sel) > 0.0
            mx23 = jnp.max(jnp.where(sel > 0, hm, NEG), axis=0, keepdims=True)
            outm23 = jnp.where(anysel, mx23, 0.0)
            outrow = outm1 + outm23
        rows.append(outrow)
        h = hnew
        noprev = no

    X = jnp.concatenate(rows, axis=0)
    tt = jnp.dot(X, wd_ref[...], preferred_element_type=f32) + bd_ref[...]
    vu = jnp.dot(tt, ctx_ref[...], preferred_element_type=f32)
    vv = vu + vneg_ref[0]
    vm = jnp.max(vv, axis=0, keepdims=True)
    ev = jnp.exp(vv - vm)
    sc = ev / jnp.sum(ev, axis=0, keepdims=True)
    pooled = jnp.sum(X * sc, axis=0, keepdims=True)
    res = jnp.dot(pooled, wc_ref[...], preferred_element_type=f32) + bc_ref[...]
    out_ref[...] = jnp.broadcast_to(res[None], (1, 8, OUT))


def _spec(arr, batched):
    nd = arr.ndim
    if batched:
        blk = (1,) + arr.shape[1:]
        return pl.BlockSpec(blk, lambda i, _n=nd: (i,) + (0,) * (_n - 1))
    return pl.BlockSpec(arr.shape, lambda i, _n=nd: (0,) * _n)


def _make_call(operands, B, OUT, body):
    in_specs = [_spec(a, j < 3) for j, a in enumerate(operands)]
    out_specs = pl.BlockSpec((1, 8, OUT), lambda i: (i, 0, 0))
    return pl.pallas_call(
        body,
        grid=(B,),
        in_specs=in_specs,
        out_specs=out_specs,
        out_shape=jax.ShapeDtypeStruct((B, 8, OUT), jnp.float32),
    )(*operands)


def kernel(code_x, divided, neighbors, lens, adj, c_emb, n_emb, u_emb,
           Wg, bg, W_ih, b_ih, W_hh, b_hh, Wq, bq, Wk, bk, Wv, bv,
           Wd, bd, ctx, Wc, bc):
    f32 = jnp.float32
    B, T, N = code_x.shape
    GS = Wg.shape[1]
    TA = Wq.shape[1]
    HS = W_hh.shape[1]
    OUT = Wc.shape[1]
    pn = NP - N
    ph = HP - HS

    adj_p = jnp.pad(adj, ((0, pn), (0, pn)))
    ce_p = jnp.pad(c_emb, ((0, pn), (0, 0)))
    ne_p = jnp.pad(n_emb, ((0, pn), (0, 0)))
    ue_p = jnp.pad(u_emb, ((0, pn), (0, 0)))

    m = (divided > 0).astype(f32)
    maskp = jnp.concatenate(
        [code_x[..., None], neighbors[..., None], m], axis=-1)
    maskp = jnp.pad(maskp, ((0, 0), (0, 0), (0, pn), (0, 3)))
    selrow_n = jnp.maximum(m[..., 1], m[..., 2])
    selrow = jnp.pad(selrow_n, ((0, 0), (0, 0), (0, pn)))

    lens_i = jnp.maximum(jnp.asarray(lens).astype(jnp.int32), 1)
    valid = jnp.arange(T)[None, :] < lens_i[:, None]
    vneg = jnp.where(valid, 0.0, NEG).astype(f32)[..., None]

    ut = (jnp.arange(NP)[:, None] <= jnp.arange(NP)[None, :]).astype(f32)

    wihT = W_ih.T
    whhT = W_hh.T
    def _split_i(k):
        return jnp.pad(wihT[:, k * HS:(k + 1) * HS], ((0, 0), (0, ph)))
    def _split_h(k):
        return jnp.pad(whhT[:, k * HS:(k + 1) * HS], ((0, ph), (0, ph)))
    def _split_b(b, k):
        return jnp.pad(b[k * HS:(k + 1) * HS], (0, ph))[None]
    wir, wiz, win = _split_i(0), _split_i(1), _split_i(2)
    whr, whz, whn = _split_h(0), _split_h(1), _split_h(2)
    bir, biz, bin_ = _split_b(b_ih, 0), _split_b(b_ih, 1), _split_b(b_ih, 2)
    bhr, bhz, bhn = _split_b(b_hh, 0), _split_b(b_hh, 1), _split_b(b_hh, 2)

    wv_p = jnp.pad(Wv, ((0, 0), (0, ph)))
    bv_p = jnp.pad(bv, (0, ph))[None]
    wd_p = jnp.pad(Wd, ((0, ph), (0, 0)))
    wc_p = jnp.pad(Wc, ((0, ph), (0, 0)))
    ctx_c = ctx[:, None]

    shared = (
        adj_p, ce_p, ne_p, ue_p,
        Wg, bg[None],
        wir, wiz, win, bir, biz, bin_,
        whr, whz, whn, bhr, bhz, bhn,
        Wq, bq[None], Wk, bk[None], wv_p, bv_p,
        wd_p, bd[None], ctx_c, wc_p, bc[None],
    )

    mk = functools.partial(_body, T=T, GS=GS, OUT=OUT,
                           inv_sqrt_ta=float(1.0 / (TA ** 0.5)))

    def _compact(ops):
        maskp_, selrow_, vneg_, ut_, rest = ops[0], ops[1], ops[2], ops[3], ops[4:]
        operands = (maskp_, selrow_, vneg_, ut_) + rest
        return _make_call(operands, B, OUT,
                          functools.partial(mk, compact=True))

    def _dense(ops):
        operands = (ops[0], ops[1], ops[2]) + ops[4:]
        return _make_call(operands, B, OUT,
                          functools.partial(mk, compact=False))

    fits = jnp.max(jnp.sum(selrow_n, axis=-1)) <= float(KC)
    out = jax.lax.cond(fits, _compact, _dense,
                       (maskp, selrow, vneg, ut) + shared)
    return out[:, 0, :]


# speedup vs baseline: 5.2134x; 1.2567x over previous
import functools

import jax
import jax.numpy as jnp
from jax.experimental import pallas as pl

NEG = -1e9
NP = 1024
HP = 152
KC = 128


def _scatter(onehot, x):
    return jax.lax.dot_general(onehot, x, (((0,), (0,)), ((), ())),
                               preferred_element_type=jnp.float32)


def _softmax_rows(g):
    gm = jnp.max(g, axis=1, keepdims=True)
    e = jnp.exp(g - gm)
    return e / jnp.sum(e, axis=1, keepdims=True)


def _body(*refs, T, GS, OUT, inv_sqrt_ta, compact):
    f32 = jnp.float32
    if compact:
        (maskp_ref, selrow_ref, m1row_ref, vneg_ref, ut_ref, adj_ref,
         ce_ref, ne_ref, ue_ref, wg_ref, bg_ref,
         wir_ref, wiz_ref, win_ref, bir_ref, biz_ref, bin_ref,
         whr_ref, whz_ref, whn_ref, bhr_ref, bhz_ref, bhn_ref,
         wq_ref, bq_ref, wk_ref, bk_ref, wv_ref, bv_ref,
         wd_ref, bd_ref, ctx_ref, wc_ref, bc_ref, out_ref) = refs
    else:
        (maskp_ref, selrow_ref, m1row_ref, vneg_ref, adj_ref,
         ce_ref, ne_ref, ue_ref, wg_ref, bg_ref,
         wir_ref, wiz_ref, win_ref, bir_ref, biz_ref, bin_ref,
         whr_ref, whz_ref, whn_ref, bhr_ref, bhz_ref, bhn_ref,
         wq_ref, bq_ref, wk_ref, bk_ref, wv_ref, bv_ref,
         wd_ref, bd_ref, ctx_ref, wc_ref, bc_ref, out_ref) = refs

    adjm = adj_ref[...]
    cemb = ce_ref[...]
    nemb = ne_ref[...]
    uemb = ue_ref[...]
    wg = wg_ref[...]; bg = bg_ref[...]
    wir = wir_ref[...]; wiz = wiz_ref[...]; win = win_ref[...]
    bir = bir_ref[...]; biz = biz_ref[...]; bin_ = bin_ref[...]
    whr = whr_ref[...]; whz = whz_ref[...]; whn = whn_ref[...]
    bhr = bhr_ref[...]; bhz = bhz_ref[...]; bhn = bhn_ref[...]
    wq = wq_ref[...]; bq = bq_ref[...]
    wk = wk_ref[...]; bk = bk_ref[...]
    wv = wv_ref[...]; bv = bv_ref[...]

    if compact:
        ut = ut_ref[...]
        srow_all = selrow_ref[0]
        m1row_all = m1row_ref[0]
        pos_sel = jnp.dot(srow_all, ut, preferred_element_type=f32)
        pos_m1 = jnp.dot(m1row_all, ut, preferred_element_type=f32)
        cnt_sel = pos_sel[:, NP - 1:NP]
        cnt_m1 = pos_m1[:, NP - 1:NP]
        kio_col = jax.lax.broadcasted_iota(jnp.int32, (KC, 1), 0).astype(f32)
        kio_row = jax.lax.broadcasted_iota(jnp.int32, (1, KC), 1).astype(f32)

    CS = cemb.shape[1]
    msk = []
    x0s = []
    for t in range(T):
        mm = maskp_ref[0, t]
        cols = (mm[:, 0:1], mm[:, 1:2], mm[:, 2:3], mm[:, 3:4], mm[:, 4:5])
        msk.append(cols)
        x0s.append(cols[0] * cemb + cols[1] * nemb)
    s_all = jnp.dot(adjm, jnp.concatenate(x0s, axis=1),
                    preferred_element_type=f32)

    h = jnp.zeros((NP, HP), f32)
    noprev = jnp.zeros((NP, GS), f32)
    rows = []
    for t in range(T):
        c, n, m1, m2, m3 = msk[t]
        s = s_all[:, t * CS:(t + 1) * CS]

        cg = jnp.dot(c * (cemb + s), wg, preferred_element_type=f32) + bg
        co = jnp.where(cg >= 0, cg, 0.01 * cg)
        ngi = jnp.dot(n * (nemb + s), wg, preferred_element_type=f32) + bg
        no = jnp.where(ngi >= 0, ngi, 0.01 * ngi)

        if compact:
            pos1_t = pos_m1[t:t + 1, :]
            oh_m1 = jnp.where(
                (pos1_t == kio_col + 1.0) & (m1row_all[t:t + 1, :] > 0),
                1.0, 0.0)
            coc = jnp.dot(oh_m1, co, preferred_element_type=f32)
            hc = jnp.dot(oh_m1, h, preferred_element_type=f32)
            gi_r = jnp.dot(coc, wir, preferred_element_type=f32) + bir
            gi_z = jnp.dot(coc, wiz, preferred_element_type=f32) + biz
            gi_n = jnp.dot(coc, win, preferred_element_type=f32) + bin_
            gh_r = jnp.dot(hc, whr, preferred_element_type=f32) + bhr
            gh_z = jnp.dot(hc, whz, preferred_element_type=f32) + bhz
            gh_n = jnp.dot(hc, whn, preferred_element_type=f32) + bhn
            r = jax.nn.sigmoid(gi_r + gh_r)
            z = jax.nn.sigmoid(gi_z + gh_z)
            ng = jnp.tanh(gi_n + r * gh_n)
            h_all_c = (1.0 - z) * ng + z * hc

            c1 = cnt_m1[t:t + 1, :]
            smask1 = kio_col < c1
            mx1 = jnp.max(jnp.where(smask1, h_all_c, NEG),
                          axis=0, keepdims=True)
            outm1 = jnp.where(c1 > 0, mx1, 0.0)

            if t == 0:
                hnew = _scatter(oh_m1, h_all_c)
                outrow = outm1
            else:
                sel = jnp.maximum(m2, m3)
                pos2_t = pos_sel[t:t + 1, :]
                oh_sel = jnp.where(
                    (pos2_t == kio_col + 1.0) & (srow_all[t:t + 1, :] > 0),
                    1.0, 0.0)
                qsrc = m2 * noprev + (1.0 - m2) * (m3 * uemb)
                qsrc_c = jnp.dot(oh_sel, qsrc, preferred_element_type=f32)
                co_c2 = jnp.dot(oh_sel, co, preferred_element_type=f32)
                queryc = jnp.dot(qsrc_c, wq, preferred_element_type=f32) + bq
                keyc = jnp.dot(qsrc_c, wk, preferred_element_type=f32) + bk
                valuec = jnp.dot(co_c2, wv, preferred_element_type=f32) + bv
                g = jax.lax.dot_general(
                    queryc, keyc, (((1,), (1,)), ((), ())),
                    preferred_element_type=f32) * inv_sqrt_ta
                c2 = cnt_sel[t:t + 1, :]
                g = jnp.where(kio_row < c2, g, NEG)
                score = _softmax_rows(g)
                hm_c = jnp.tanh(
                    jnp.dot(score, valuec, preferred_element_type=f32))
                smask2 = kio_col < c2
                mx23 = jnp.max(jnp.where(smask2, hm_c, NEG),
                               axis=0, keepdims=True)
                outm23 = jnp.where(c2 > 0, mx23, 0.0)
                selatm1 = jnp.dot(oh_m1, sel, preferred_element_type=f32)
                hnew = (_scatter(oh_m1, h_all_c * (1.0 - selatm1)) +
                        _scatter(oh_sel, hm_c))
                outrow = outm1 + outm23
        else:
            gi_r = jnp.dot(co, wir, preferred_element_type=f32) + bir
            gi_z = jnp.dot(co, wiz, preferred_element_type=f32) + biz
            gi_n = jnp.dot(co, win, preferred_element_type=f32) + bin_
            gh_r = jnp.dot(h, whr, preferred_element_type=f32) + bhr
            gh_z = jnp.dot(h, whz, preferred_element_type=f32) + bhz
            gh_n = jnp.dot(h, whn, preferred_element_type=f32) + bhn
            r = jax.nn.sigmoid(gi_r + gh_r)
            z = jax.nn.sigmoid(gi_z + gh_z)
            ng = jnp.tanh(gi_n + r * gh_n)
            h_all = (1.0 - z) * ng + z * h

            hnew = m1 * h_all
            anym1 = jnp.max(m1) > 0.0
            mx1 = jnp.max(jnp.where(m1 > 0, h_all, NEG),
                          axis=0, keepdims=True)
            outm1 = jnp.where(anym1, mx1, 0.0)

            if t == 0:
                outrow = outm1
            else:
                sel = jnp.maximum(m2, m3)
                srow = selrow_ref[0, t:t + 1, :]
                qsrc = m2 * noprev + (1.0 - m2) * (m3 * uemb)
                query = jnp.dot(qsrc, wq, preferred_element_type=f32) + bq
                key = jnp.dot(qsrc, wk, preferred_element_type=f32) + bk
                value = jnp.dot(co, wv, preferred_element_type=f32) + bv
                g = jax.lax.dot_general(
                    query, key, (((1,), (1,)), ((), ())),
                    preferred_element_type=f32) * inv_sqrt_ta
                g = jnp.where(srow > 0, g, NEG)
                score = _softmax_rows(g)
                hm = jnp.tanh(
                    jnp.dot(score, value, preferred_element_type=f32))
                hnew = sel * hm + (1.0 - sel) * hnew
                anysel = jnp.max(sel) > 0.0
                mx23 = jnp.max(jnp.where(sel > 0, hm, NEG),
                               axis=0, keepdims=True)
                outm23 = jnp.where(anysel, mx23, 0.0)
                outrow = outm1 + outm23
        rows.append(outrow)
        h = hnew
        noprev = no

    X = jnp.concatenate(rows, axis=0)
    tt = jnp.dot(X, wd_ref[...], preferred_element_type=f32) + bd_ref[...]
    vu = jnp.dot(tt, ctx_ref[...], preferred_element_type=f32)
    vv = vu + vneg_ref[0]
    vm = jnp.max(vv, axis=0, keepdims=True)
    ev = jnp.exp(vv - vm)
    sc = ev / jnp.sum(ev, axis=0, keepdims=True)
    pooled = jnp.sum(X * sc, axis=0, keepdims=True)
    res = jnp.dot(pooled, wc_ref[...], preferred_element_type=f32) + bc_ref[...]
    out_ref[...] = jnp.broadcast_to(res[None], (1, 8, OUT))


def _spec(arr, batched):
    nd = arr.ndim
    if batched:
        blk = (1,) + arr.shape[1:]
        return pl.BlockSpec(blk, lambda i, _n=nd: (i,) + (0,) * (_n - 1))
    return pl.BlockSpec(arr.shape, lambda i, _n=nd: (0,) * _n)


def _make_call(operands, B, OUT, body, n_batched):
    in_specs = [_spec(a, j < n_batched) for j, a in enumerate(operands)]
    out_specs = pl.BlockSpec((1, 8, OUT), lambda i: (i, 0, 0))
    return pl.pallas_call(
        body,
        grid=(B,),
        in_specs=in_specs,
        out_specs=out_specs,
        out_shape=jax.ShapeDtypeStruct((B, 8, OUT), jnp.float32),
    )(*operands)


def kernel(code_x, divided, neighbors, lens, adj, c_emb, n_emb, u_emb,
           Wg, bg, W_ih, b_ih, W_hh, b_hh, Wq, bq, Wk, bk, Wv, bv,
           Wd, bd, ctx, Wc, bc):
    f32 = jnp.float32
    B, T, N = code_x.shape
    GS = Wg.shape[1]
    TA = Wq.shape[1]
    HS = W_hh.shape[1]
    OUT = Wc.shape[1]
    pn = NP - N
    ph = HP - HS

    adj_p = jnp.pad(adj, ((0, pn), (0, pn)))
    ce_p = jnp.pad(c_emb, ((0, pn), (0, 0)))
    ne_p = jnp.pad(n_emb, ((0, pn), (0, 0)))
    ue_p = jnp.pad(u_emb, ((0, pn), (0, 0)))

    m = (divided > 0).astype(f32)
    maskp = jnp.concatenate(
        [code_x[..., None], neighbors[..., None], m], axis=-1)
    maskp = jnp.pad(maskp, ((0, 0), (0, 0), (0, pn), (0, 3)))
    selrow_n = jnp.maximum(m[..., 1], m[..., 2])
    selrow = jnp.pad(selrow_n, ((0, 0), (0, 0), (0, pn)))
    m1row = jnp.pad(m[..., 0], ((0, 0), (0, 0), (0, pn)))

    lens_i = jnp.maximum(jnp.asarray(lens).astype(jnp.int32), 1)
    valid = jnp.arange(T)[None, :] < lens_i[:, None]
    vneg = jnp.where(valid, 0.0, NEG).astype(f32)[..., None]

    ut = (jnp.arange(NP)[:, None] <= jnp.arange(NP)[None, :]).astype(f32)

    wihT = W_ih.T
    whhT = W_hh.T
    def _split_i(k):
        return jnp.pad(wihT[:, k * HS:(k + 1) * HS], ((0, 0), (0, ph)))
    def _split_h(k):
        return jnp.pad(whhT[:, k * HS:(k + 1) * HS], ((0, ph), (0, ph)))
    def _split_b(b, k):
        return jnp.pad(b[k * HS:(k + 1) * HS], (0, ph))[None]
    wir, wiz, win = _split_i(0), _split_i(1), _split_i(2)
    whr, whz, whn = _split_h(0), _split_h(1), _split_h(2)
    bir, biz, bin_ = _split_b(b_ih, 0), _split_b(b_ih, 1), _split_b(b_ih, 2)
    bhr, bhz, bhn = _split_b(b_hh, 0), _split_b(b_hh, 1), _split_b(b_hh, 2)

    wv_p = jnp.pad(Wv, ((0, 0), (0, ph)))
    bv_p = jnp.pad(bv, (0, ph))[None]
    wd_p = jnp.pad(Wd, ((0, ph), (0, 0)))
    wc_p = jnp.pad(Wc, ((0, ph), (0, 0)))
    ctx_c = ctx[:, None]

    shared = (
        adj_p, ce_p, ne_p, ue_p,
        Wg, bg[None],
        wir, wiz, win, bir, biz, bin_,
        whr, whz, whn, bhr, bhz, bhn,
        Wq, bq[None], Wk, bk[None], wv_p, bv_p,
        wd_p, bd[None], ctx_c, wc_p, bc[None],
    )

    mk = functools.partial(_body, T=T, GS=GS, OUT=OUT,
                           inv_sqrt_ta=float(1.0 / (TA ** 0.5)))

    def _compact(ops):
        operands = ops[:4] + (ops[4],) + ops[5:]
        return _make_call(operands, B, OUT,
                          functools.partial(mk, compact=True), 4)

    def _dense(ops):
        operands = ops[:4] + ops[5:]
        return _make_call(operands, B, OUT,
                          functools.partial(mk, compact=False), 4)

    fits = jnp.maximum(jnp.max(jnp.sum(selrow_n, axis=-1)),
                       jnp.max(jnp.sum(m[..., 0], axis=-1))) <= float(KC)
    out = jax.lax.cond(fits, _compact, _dense,
                       (maskp, selrow, m1row, vneg, ut) + shared)
    return out[:, 0, :]


# speedup vs baseline: 5.3721x; 1.0304x over previous
import functools

import jax
import jax.numpy as jnp
from jax.experimental import pallas as pl

NEG = -1e9
NP = 1024
HP = 152
KC = 128


def _scatter(onehot, x):
    return jax.lax.dot_general(onehot, x, (((0,), (0,)), ((), ())),
                               preferred_element_type=jnp.float32)


def _softmax_rows(g):
    gm = jnp.max(g, axis=1, keepdims=True)
    e = jnp.exp(g - gm)
    return e / jnp.sum(e, axis=1, keepdims=True)


def _body(*refs, T, GS, OUT, inv_sqrt_ta, compact):
    f32 = jnp.float32
    if compact:
        (maskp_ref, selrow_ref, m1row_ref, vneg_ref, ut_ref, adj_ref,
         ce_ref, ne_ref, ue_ref, wg_ref, bg_ref,
         wir_ref, wiz_ref, win_ref, bir_ref, biz_ref, bin_ref,
         whr_ref, whz_ref, whn_ref, bhr_ref, bhz_ref, bhn_ref,
         wq_ref, bq_ref, wk_ref, bk_ref, wv_ref, bv_ref,
         wd_ref, bd_ref, ctx_ref, wc_ref, bc_ref, out_ref) = refs
    else:
        (maskp_ref, selrow_ref, m1row_ref, vneg_ref, adj_ref,
         ce_ref, ne_ref, ue_ref, wg_ref, bg_ref,
         wir_ref, wiz_ref, win_ref, bir_ref, biz_ref, bin_ref,
         whr_ref, whz_ref, whn_ref, bhr_ref, bhz_ref, bhn_ref,
         wq_ref, bq_ref, wk_ref, bk_ref, wv_ref, bv_ref,
         wd_ref, bd_ref, ctx_ref, wc_ref, bc_ref, out_ref) = refs

    adjm = adj_ref[...]
    cemb = ce_ref[...]
    nemb = ne_ref[...]
    uemb = ue_ref[...]
    wg = wg_ref[...]; bg = bg_ref[...]
    wir = wir_ref[...]; wiz = wiz_ref[...]; win = win_ref[...]
    bir = bir_ref[...]; biz = biz_ref[...]; bin_ = bin_ref[...]
    whr = whr_ref[...]; whz = whz_ref[...]; whn = whn_ref[...]
    bhr = bhr_ref[...]; bhz = bhz_ref[...]; bhn = bhn_ref[...]
    wq = wq_ref[...]; bq = bq_ref[...]
    wk = wk_ref[...]; bk = bk_ref[...]
    wv = wv_ref[...]; bv = bv_ref[...]

    if compact:
        ut = ut_ref[...]
        srow_all = selrow_ref[0]
        m1row_all = m1row_ref[0]
        pos_sel = jnp.dot(srow_all, ut, preferred_element_type=f32)
        pos_m1 = jnp.dot(m1row_all, ut, preferred_element_type=f32)
        cnt_sel = pos_sel[:, NP - 1:NP]
        cnt_m1 = pos_m1[:, NP - 1:NP]
        kio_col = jax.lax.broadcasted_iota(jnp.int32, (KC, 1), 0).astype(f32)
        kio_row = jax.lax.broadcasted_iota(jnp.int32, (1, KC), 1).astype(f32)
        iota_np = jax.lax.broadcasted_iota(jnp.int32, (NP, 1), 0).astype(f32)

    CS = cemb.shape[1]
    msk = []
    x0s = []
    for t in range(T):
        mm = maskp_ref[0, t]
        cols = (mm[:, 0:1], mm[:, 1:2], mm[:, 2:3], mm[:, 3:4], mm[:, 4:5])
        msk.append(cols)
        x0s.append(cols[0] * cemb + cols[1] * nemb)
    s_all = jnp.dot(adjm, jnp.concatenate(x0s, axis=1),
                    preferred_element_type=f32)

    h = jnp.zeros((NP, HP), f32)
    noprev = jnp.zeros((NP, GS), f32)
    hparts = []
    rows = []
    for t in range(T):
        c, n, m1, m2, m3 = msk[t]
        s = s_all[:, t * CS:(t + 1) * CS]

        cg = jnp.dot(c * (cemb + s), wg, preferred_element_type=f32) + bg
        co = jnp.where(cg >= 0, cg, 0.01 * cg)
        ngi = jnp.dot(n * (nemb + s), wg, preferred_element_type=f32) + bg
        no = jnp.where(ngi >= 0, ngi, 0.01 * ngi)

        if compact:
            pos1_t = pos_m1[t:t + 1, :]
            oh_m1 = jnp.where(
                (pos1_t == kio_col + 1.0) & (m1row_all[t:t + 1, :] > 0),
                1.0, 0.0)
            c1 = cnt_m1[t:t + 1, :]
            idx1_col = jnp.dot(oh_m1, iota_np,
                               preferred_element_type=f32)
            coc = jnp.dot(oh_m1, co, preferred_element_type=f32)
            if hparts:
                hc = None
                for (pidx, pcnt, pval) in hparts:
                    tr = jnp.where(
                        (idx1_col == pidx) & (kio_col < c1) & (kio_row < pcnt),
                        1.0, 0.0)
                    piece = jnp.dot(tr, pval, preferred_element_type=f32)
                    hc = piece if hc is None else hc + piece
            else:
                hc = jnp.zeros((KC, HP), f32)
            gi_r = jnp.dot(coc, wir, preferred_element_type=f32) + bir
            gi_z = jnp.dot(coc, wiz, preferred_element_type=f32) + biz
            gi_n = jnp.dot(coc, win, preferred_element_type=f32) + bin_
            gh_r = jnp.dot(hc, whr, preferred_element_type=f32) + bhr
            gh_z = jnp.dot(hc, whz, preferred_element_type=f32) + bhz
            gh_n = jnp.dot(hc, whn, preferred_element_type=f32) + bhn
            r = jax.nn.sigmoid(gi_r + gh_r)
            z = jax.nn.sigmoid(gi_z + gh_z)
            ng = jnp.tanh(gi_n + r * gh_n)
            h_all_c = (1.0 - z) * ng + z * hc

            smask1 = kio_col < c1
            mx1 = jnp.max(jnp.where(smask1, h_all_c, NEG),
                          axis=0, keepdims=True)
            outm1 = jnp.where(c1 > 0, mx1, 0.0)

            idx1_row = jax.lax.dot_general(
                iota_np, oh_m1, (((0,), (1,)), ((), ())),
                preferred_element_type=f32)
            if t == 0:
                hparts = [(idx1_row, c1, h_all_c)]
                outrow = outm1
            else:
                sel = jnp.maximum(m2, m3)
                pos2_t = pos_sel[t:t + 1, :]
                oh_sel = jnp.where(
                    (pos2_t == kio_col + 1.0) & (srow_all[t:t + 1, :] > 0),
                    1.0, 0.0)
                qsrc = m2 * noprev + (1.0 - m2) * (m3 * uemb)
                qsrc_c = jnp.dot(oh_sel, qsrc, preferred_element_type=f32)
                co_c2 = jnp.dot(oh_sel, co, preferred_element_type=f32)
                queryc = jnp.dot(qsrc_c, wq, preferred_element_type=f32) + bq
                keyc = jnp.dot(qsrc_c, wk, preferred_element_type=f32) + bk
                valuec = jnp.dot(co_c2, wv, preferred_element_type=f32) + bv
                g = jax.lax.dot_general(
                    queryc, keyc, (((1,), (1,)), ((), ())),
                    preferred_element_type=f32) * inv_sqrt_ta
                c2 = cnt_sel[t:t + 1, :]
                g = jnp.where(kio_row < c2, g, NEG)
                score = _softmax_rows(g)
                hm_c = jnp.tanh(
                    jnp.dot(score, valuec, preferred_element_type=f32))
                smask2 = kio_col < c2
                mx23 = jnp.max(jnp.where(smask2, hm_c, NEG),
                               axis=0, keepdims=True)
                outm23 = jnp.where(c2 > 0, mx23, 0.0)
                selatm1 = jnp.dot(oh_m1, sel, preferred_element_type=f32)
                idx2_row = jax.lax.dot_general(
                    iota_np, oh_sel, (((0,), (1,)), ((), ())),
                    preferred_element_type=f32)
                hparts = [(idx1_row, c1, h_all_c * (1.0 - selatm1)),
                          (idx2_row, c2, hm_c)]
                outrow = outm1 + outm23
        else:
            gi_r = jnp.dot(co, wir, preferred_element_type=f32) + bir
            gi_z = jnp.dot(co, wiz, preferred_element_type=f32) + biz
            gi_n = jnp.dot(co, win, preferred_element_type=f32) + bin_
            gh_r = jnp.dot(h, whr, preferred_element_type=f32) + bhr
            gh_z = jnp.dot(h, whz, preferred_element_type=f32) + bhz
            gh_n = jnp.dot(h, whn, preferred_element_type=f32) + bhn
            r = jax.nn.sigmoid(gi_r + gh_r)
            z = jax.nn.sigmoid(gi_z + gh_z)
            ng = jnp.tanh(gi_n + r * gh_n)
            h_all = (1.0 - z) * ng + z * h

            hnew = m1 * h_all
            anym1 = jnp.max(m1) > 0.0
            mx1 = jnp.max(jnp.where(m1 > 0, h_all, NEG),
                          axis=0, keepdims=True)
            outm1 = jnp.where(anym1, mx1, 0.0)

            if t == 0:
                outrow = outm1
            else:
                sel = jnp.maximum(m2, m3)
                srow = selrow_ref[0, t:t + 1, :]
                qsrc = m2 * noprev + (1.0 - m2) * (m3 * uemb)
                query = jnp.dot(qsrc, wq, preferred_element_type=f32) + bq
                key = jnp.dot(qsrc, wk, preferred_element_type=f32) + bk
                value = jnp.dot(co, wv, preferred_element_type=f32) + bv
                g = jax.lax.dot_general(
                    query, key, (((1,), (1,)), ((), ())),
                    preferred_element_type=f32) * inv_sqrt_ta
                g = jnp.where(srow > 0, g, NEG)
                score = _softmax_rows(g)
                hm = jnp.tanh(
                    jnp.dot(score, value, preferred_element_type=f32))
                hnew = sel * hm + (1.0 - sel) * hnew
                anysel = jnp.max(sel) > 0.0
                mx23 = jnp.max(jnp.where(sel > 0, hm, NEG),
                               axis=0, keepdims=True)
                outm23 = jnp.where(anysel, mx23, 0.0)
                outrow = outm1 + outm23
        rows.append(outrow)
        if not compact:
            h = hnew
        noprev = no

    X = jnp.concatenate(rows, axis=0)
    tt = jnp.dot(X, wd_ref[...], preferred_element_type=f32) + bd_ref[...]
    vu = jnp.dot(tt, ctx_ref[...], preferred_element_type=f32)
    vv = vu + vneg_ref[0]
    vm = jnp.max(vv, axis=0, keepdims=True)
    ev = jnp.exp(vv - vm)
    sc = ev / jnp.sum(ev, axis=0, keepdims=True)
    pooled = jnp.sum(X * sc, axis=0, keepdims=True)
    res = jnp.dot(pooled, wc_ref[...], preferred_element_type=f32) + bc_ref[...]
    out_ref[...] = jnp.broadcast_to(res[None], (1, 8, OUT))


def _spec(arr, batched):
    nd = arr.ndim
    if batched:
        blk = (1,) + arr.shape[1:]
        return pl.BlockSpec(blk, lambda i, _n=nd: (i,) + (0,) * (_n - 1))
    return pl.BlockSpec(arr.shape, lambda i, _n=nd: (0,) * _n)


def _make_call(operands, B, OUT, body, n_batched):
    in_specs = [_spec(a, j < n_batched) for j, a in enumerate(operands)]
    out_specs = pl.BlockSpec((1, 8, OUT), lambda i: (i, 0, 0))
    return pl.pallas_call(
        body,
        grid=(B,),
        in_specs=in_specs,
        out_specs=out_specs,
        out_shape=jax.ShapeDtypeStruct((B, 8, OUT), jnp.float32),
    )(*operands)


def kernel(code_x, divided, neighbors, lens, adj, c_emb, n_emb, u_emb,
           Wg, bg, W_ih, b_ih, W_hh, b_hh, Wq, bq, Wk, bk, Wv, bv,
           Wd, bd, ctx, Wc, bc):
    f32 = jnp.float32
    B, T, N = code_x.shape
    GS = Wg.shape[1]
    TA = Wq.shape[1]
    HS = W_hh.shape[1]
    OUT = Wc.shape[1]
    pn = NP - N
    ph = HP - HS

    adj_p = jnp.pad(adj, ((0, pn), (0, pn)))
    ce_p = jnp.pad(c_emb, ((0, pn), (0, 0)))
    ne_p = jnp.pad(n_emb, ((0, pn), (0, 0)))
    ue_p = jnp.pad(u_emb, ((0, pn), (0, 0)))

    m = (divided > 0).astype(f32)
    maskp = jnp.concatenate(
        [code_x[..., None], neighbors[..., None], m], axis=-1)
    maskp = jnp.pad(maskp, ((0, 0), (0, 0), (0, pn), (0, 3)))
    selrow_n = jnp.maximum(m[..., 1], m[..., 2])
    selrow = jnp.pad(selrow_n, ((0, 0), (0, 0), (0, pn)))
    m1row = jnp.pad(m[..., 0], ((0, 0), (0, 0), (0, pn)))

    lens_i = jnp.maximum(jnp.asarray(lens).astype(jnp.int32), 1)
    valid = jnp.arange(T)[None, :] < lens_i[:, None]
    vneg = jnp.where(valid, 0.0, NEG).astype(f32)[..., None]

    ut = (jnp.arange(NP)[:, None] <= jnp.arange(NP)[None, :]).astype(f32)

    wihT = W_ih.T
    whhT = W_hh.T
    def _split_i(k):
        return jnp.pad(wihT[:, k * HS:(k + 1) * HS], ((0, 0), (0, ph)))
    def _split_h(k):
        return jnp.pad(whhT[:, k * HS:(k + 1) * HS], ((0, ph), (0, ph)))
    def _split_b(b, k):
        return jnp.pad(b[k * HS:(k + 1) * HS], (0, ph))[None]
    wir, wiz, win = _split_i(0), _split_i(1), _split_i(2)
    whr, whz, whn = _split_h(0), _split_h(1), _split_h(2)
    bir, biz, bin_ = _split_b(b_ih, 0), _split_b(b_ih, 1), _split_b(b_ih, 2)
    bhr, bhz, bhn = _split_b(b_hh, 0), _split_b(b_hh, 1), _split_b(b_hh, 2)

    wv_p = jnp.pad(Wv, ((0, 0), (0, ph)))
    bv_p = jnp.pad(bv, (0, ph))[None]
    wd_p = jnp.pad(Wd, ((0, ph), (0, 0)))
    wc_p = jnp.pad(Wc, ((0, ph), (0, 0)))
    ctx_c = ctx[:, None]

    shared = (
        adj_p, ce_p, ne_p, ue_p,
        Wg, bg[None],
        wir, wiz, win, bir, biz, bin_,
        whr, whz, whn, bhr, bhz, bhn,
        Wq, bq[None], Wk, bk[None], wv_p, bv_p,
        wd_p, bd[None], ctx_c, wc_p, bc[None],
    )

    mk = functools.partial(_body, T=T, GS=GS, OUT=OUT,
                           inv_sqrt_ta=float(1.0 / (TA ** 0.5)))

    def _compact(ops):
        operands = ops[:4] + (ops[4],) + ops[5:]
        return _make_call(operands, B, OUT,
                          functools.partial(mk, compact=True), 4)

    def _dense(ops):
        operands = ops[:4] + ops[5:]
        return _make_call(operands, B, OUT,
                          functools.partial(mk, compact=False), 4)

    fits = jnp.maximum(jnp.max(jnp.sum(selrow_n, axis=-1)),
                       jnp.max(jnp.sum(m[..., 0], axis=-1))) <= float(KC)
    out = jax.lax.cond(fits, _compact, _dense,
                       (maskp, selrow, m1row, vneg, ut) + shared)
    return out[:, 0, :]
